# baseline (speedup 1.0000x reference)
import functools

import jax
import jax.numpy as jnp
from jax import lax
from jax.experimental import pallas as pl
from jax.experimental.pallas import tpu as pltpu
from jax.experimental.pallas import tpu_sc as plsc

_NUM_EXPERTS = 8
_TOP_K = 2
_BM = 256
_NC, _NS = 2, 16
_NW = _NC * _NS
_CHUNK = 16
_CCH = 8
_LANES = 16


def _sc_dispatch(hidden_flat, pos0, pos1, n_tok, p, d, dtype):
    tok_w = n_tok // _NW
    nch = tok_w // _CHUNK

    mesh = plsc.VectorSubcoreMesh(core_axis_name="c", subcore_axis_name="s")

    @functools.partial(
        pl.kernel,
        out_type=jax.ShapeDtypeStruct((p, d), dtype),
        mesh=mesh,
        scratch_types=[
            pltpu.VMEM((nch, _CHUNK), jnp.int32),
            pltpu.VMEM((nch, _CHUNK), jnp.int32),
            pltpu.VMEM((_CHUNK, d), dtype),
            pltpu.VMEM((_CHUNK, d), dtype),
            pltpu.SemaphoreType.DMA,
            pltpu.SemaphoreType.DMA,
            pltpu.SemaphoreType.DMA,
        ],
    )
    def k(hid_hbm, p0_hbm, p1_hbm, out_hbm, i0_v, i1_v, buf0, buf1,
          sem0, sem1, semw):
        wid = lax.axis_index("s") * _NC + lax.axis_index("c")
        base = wid * tok_w
        pltpu.sync_copy(p0_hbm.at[wid], i0_v)
        pltpu.sync_copy(p1_hbm.at[wid], i1_v)

        def rd(c, buf, sem):
            pltpu.async_copy(hid_hbm.at[pl.ds(base + c * _CHUNK, _CHUNK)],
                             buf, sem)

        def rd_wait(buf, sem):
            pltpu.make_async_copy(hid_hbm.at[pl.ds(0, _CHUNK)], buf,
                                  sem).wait()

        def wr(c, buf):
            pltpu.async_copy(buf, out_hbm.at[i0_v.at[c]], semw)
            pltpu.async_copy(buf, out_hbm.at[i1_v.at[c]], semw)

        def wr_wait(buf):
            pltpu.make_async_copy(buf, out_hbm.at[i0_v.at[0]], semw).wait()
            pltpu.make_async_copy(buf, out_hbm.at[i0_v.at[0]], semw).wait()

        rd(0, buf0, sem0)

        def body(i, _):
            e = 2 * i
            rd(e + 1, buf1, sem1)
            rd_wait(buf0, sem0)
            wr(e, buf0)
            wr_wait(buf0)

            @pl.when(e + 2 < nch)
            def _():
                rd(e + 2, buf0, sem0)

            rd_wait(buf1, sem1)
            wr(e + 1, buf1)
            wr_wait(buf1)
            return 0

        lax.fori_loop(0, nch // 2, body, 0)

    return k(hidden_flat, pos0.reshape(_NW, nch, _CHUNK),
             pos1.reshape(_NW, nch, _CHUNK))


def _sc_combine(y, pos0, pos1, wexp0, wexp1, n_tok, d):
    tok_w = n_tok // _NW
    nch = tok_w // _CCH
    nvec = d // _LANES

    mesh = plsc.VectorSubcoreMesh(core_axis_name="c", subcore_axis_name="s")

    @functools.partial(
        pl.kernel,
        out_type=jax.ShapeDtypeStruct((n_tok, d), jnp.float32),
        mesh=mesh,
        scratch_types=[
            pltpu.VMEM((tok_w,), jnp.int32),
            pltpu.VMEM((tok_w,), jnp.int32),
            pltpu.VMEM((tok_w, _LANES), jnp.float32),
            pltpu.VMEM((tok_w, _LANES), jnp.float32),
            pltpu.VMEM((_CCH, d), jnp.float32),
            pltpu.VMEM((_CCH, d), jnp.float32),
            pltpu.VMEM((_CCH, d), jnp.float32),
            pltpu.VMEM((_CCH, d), jnp.float32),
            pltpu.SemaphoreType.DMA,
            pltpu.SemaphoreType.DMA,
            pltpu.SemaphoreType.DMA,
        ],
    )
    def k(y_hbm, p0_hbm, p1_hbm, w0_hbm, w1_hbm, out_hbm,
          i0_v, i1_v, wa_v, wb_v, a0, b0, a1, b1, sem0, sem1, semw):
        wid = lax.axis_index("s") * _NC + lax.axis_index("c")
        base = wid * tok_w
        pltpu.sync_copy(p0_hbm.at[pl.ds(base, tok_w)], i0_v)
        pltpu.sync_copy(p1_hbm.at[pl.ds(base, tok_w)], i1_v)
        pltpu.sync_copy(w0_hbm.at[pl.ds(base, tok_w)], wa_v)
        pltpu.sync_copy(w1_hbm.at[pl.ds(base, tok_w)], wb_v)

        def gath(c, abuf, bbuf, sem):
            pltpu.async_copy(y_hbm.at[i0_v.at[pl.ds(c * _CCH, _CCH)]],
                             abuf, sem)
            pltpu.async_copy(y_hbm.at[i1_v.at[pl.ds(c * _CCH, _CCH)]],
                             bbuf, sem)

        def gath_wait(abuf, bbuf, sem):
            pltpu.make_async_copy(y_hbm.at[i0_v.at[pl.ds(0, _CCH)]],
                                  abuf, sem).wait()
            pltpu.make_async_copy(y_hbm.at[i0_v.at[pl.ds(0, _CCH)]],
                                  bbuf, sem).wait()

        def comp(c, abuf, bbuf):
            was = [wa_v[c * _CCH + r, :] for r in range(_CCH)]
            wbs = [wb_v[c * _CCH + r, :] for r in range(_CCH)]

            def vbody(j, _):
                col = j * _LANES
                for r in range(_CCH):
                    av = abuf[r, pl.ds(col, _LANES)]
                    bv = bbuf[r, pl.ds(col, _LANES)]
                    abuf[r, pl.ds(col, _LANES)] = was[r] * av + wbs[r] * bv
                return 0

            lax.fori_loop(0, nvec, vbody, 0)

        def wrt(c, abuf):
            pltpu.async_copy(abuf, out_hbm.at[pl.ds(base + c * _CCH, _CCH)],
                             semw)

        def wrt_wait(abuf):
            pltpu.make_async_copy(a0, out_hbm.at[pl.ds(0, _CCH)], semw).wait()

        gath(0, a0, b0, sem0)
        gath(1, a1, b1, sem1)

        def body(i, _):
            e = 2 * i
            gath_wait(a0, b0, sem0)
            comp(e, a0, b0)
            wrt(e, a0)

            @pl.when(e + 2 < nch)
            def _():
                wrt_wait(a0)
                gath(e + 2, a0, b0, sem0)

            gath_wait(a1, b1, sem1)
            comp(e + 1, a1, b1)
            wrt(e + 1, a1)

            @pl.when(e + 3 < nch)
            def _():
                wrt_wait(a1)
                gath(e + 3, a1, b1, sem1)
            return 0

        lax.fori_loop(0, nch // 2, body, 0)

        wrt_wait(a0)
        wrt_wait(a1)

    return k(y, pos0, pos1, wexp0, wexp1)


def _matmul_block(be_ref, ff_ref, slot_ref, nxt_ref, x_ref, w_hbm, b_ref,
                  o_ref, wbuf, sems):
    i = pl.program_id(0)
    e_raw = be_ref[i]
    slot = slot_ref[i]

    @pl.when(i == 0)
    def _():
        pltpu.make_async_copy(w_hbm.at[be_ref[0]], wbuf.at[0],
                              sems.at[0]).start()

        @pl.when(nxt_ref[0] < _NUM_EXPERTS)
        def _():
            pltpu.make_async_copy(w_hbm.at[nxt_ref[0]], wbuf.at[1],
                                  sems.at[1]).start()

    @pl.when(ff_ref[i] == 1)
    def _():
        pltpu.make_async_copy(w_hbm.at[0], wbuf.at[slot],
                              sems.at[slot]).wait()

        @pl.when((i > 0) & (nxt_ref[i] < _NUM_EXPERTS))
        def _():
            pltpu.make_async_copy(w_hbm.at[nxt_ref[i]], wbuf.at[1 - slot],
                                  sems.at[1 - slot]).start()

    @pl.when(e_raw < _NUM_EXPERTS)
    def _():
        x = x_ref[...]
        w = wbuf[slot].astype(jnp.bfloat16)
        y = lax.dot_general(x, w, (((1,), (1,)), ((), ())),
                            preferred_element_type=jnp.float32)
        o_ref[...] = y + b_ref[0]


def _grouped_matmul(dispatch, W, b, block_expert, num_blocks, d):
    e = _NUM_EXPERTS
    be = block_expert
    ff = ((be != jnp.concatenate([jnp.full((1,), 9, jnp.int32), be[:-1]]))
          & (be < e)).astype(jnp.int32)
    slot = ((jnp.cumsum(ff) - 1) % 2).astype(jnp.int32)
    nxt = jnp.min(jnp.where(be[None, :] > be[:, None], be[None, :], e),
                  axis=1).astype(jnp.int32)

    grid_spec = pltpu.PrefetchScalarGridSpec(
        num_scalar_prefetch=4,
        grid=(num_blocks,),
        in_specs=[
            pl.BlockSpec((_BM, d), lambda i, *_: (i, 0)),
            pl.BlockSpec(memory_space=pl.ANY),
            pl.BlockSpec(
                (1, 1, d),
                lambda i, be_, *_: (jnp.minimum(be_[i], e - 1), 0, 0)),
        ],
        out_specs=pl.BlockSpec((_BM, d), lambda i, *_: (i, 0)),
        scratch_shapes=[
            pltpu.VMEM((2, d, d), jnp.float32),
            pltpu.SemaphoreType.DMA((2,)),
        ],
    )
    return pl.pallas_call(
        _matmul_block,
        grid_spec=grid_spec,
        out_shape=jax.ShapeDtypeStruct((num_blocks * _BM, d), jnp.float32),
    )(be, ff, slot, nxt, dispatch, W, b.reshape(b.shape[0], 1, d))


def kernel(hidden, gate_logits, W, b):
    bsz, seq, d = hidden.shape
    n_tok = bsz * seq
    k = _TOP_K
    e = _NUM_EXPERTS
    n_slots = n_tok * k
    p = n_slots + e * _BM
    num_blocks = p // _BM

    hidden_flat = hidden.reshape(n_tok, d)

    g_t = gate_logits.T
    eids = jnp.arange(e, dtype=jnp.int32)[:, None]
    gmax = jnp.max(g_t, axis=0, keepdims=True)
    ex = jnp.exp(g_t - gmax)
    denom = jnp.sum(ex, axis=0, keepdims=True)
    probs_t = ex / denom
    v1 = jnp.max(probs_t, axis=0, keepdims=True)
    e1 = jnp.min(jnp.where(probs_t == v1, eids, e), axis=0)
    probs_m = jnp.where(eids == e1[None, :], -jnp.inf, probs_t)
    v2 = jnp.max(probs_m, axis=0, keepdims=True)
    e2 = jnp.min(jnp.where(probs_m == v2, eids, e), axis=0)

    flat_e = jnp.stack([e1, e2], axis=1).reshape(1, -1)
    oh_t = (eids == flat_e).astype(jnp.int32)
    cum_t = jnp.cumsum(oh_t, axis=1)
    rank = jnp.sum(oh_t * cum_t, axis=0) - 1
    counts = cum_t[:, -1]
    padded_counts = ((counts + _BM - 1) // _BM) * _BM
    padded_end = jnp.cumsum(padded_counts)
    padded_start = padded_end - padded_counts
    seg_base = jnp.sum(oh_t * padded_start[:, None], axis=0)
    padded_pos = (seg_base + rank).astype(jnp.int32)

    block_expert = jnp.sum(
        jnp.arange(num_blocks)[:, None] * _BM >= padded_end[None, :],
        axis=1).astype(jnp.int32)

    pos = padded_pos.reshape(n_tok, k)
    pos0, pos1 = pos[:, 0], pos[:, 1]
    wexp0 = jnp.broadcast_to(v1.reshape(n_tok, 1), (n_tok, _LANES))
    wexp1 = jnp.broadcast_to(v2.reshape(n_tok, 1), (n_tok, _LANES))

    hid16_i32 = lax.bitcast_convert_type(
        hidden_flat.astype(jnp.bfloat16).reshape(n_tok, d // 2, 2),
        jnp.int32)
    disp_i32 = _sc_dispatch(hid16_i32, pos0, pos1, n_tok, p, d // 2,
                            jnp.int32)
    dispatch = lax.bitcast_convert_type(
        disp_i32, jnp.bfloat16).reshape(p, d)

    y = _grouped_matmul(dispatch, W, b, block_expert, num_blocks, d)

    combined = _sc_combine(y, pos0, pos1, wexp0, wexp1, n_tok, d)
    return combined.reshape(bsz, seq, d)


# speedup vs baseline: 3.0842x; 3.0842x over previous
import functools

import jax
import jax.numpy as jnp
from jax import lax
from jax.experimental import pallas as pl
from jax.experimental.pallas import tpu as pltpu
from jax.experimental.pallas import tpu_sc as plsc

_NUM_EXPERTS = 8
_TOP_K = 2
_BM = 256
_NC, _NS = 2, 16
_NW = _NC * _NS
_CHUNK = 16
_CCH = 8
_LANES = 16


def _sc_dispatch(hidden_flat, pos0, pos1, n_tok, p, d, dtype):
    tok_w = n_tok // _NW
    nch = tok_w // _CHUNK

    mesh = plsc.VectorSubcoreMesh(core_axis_name="c", subcore_axis_name="s")

    @functools.partial(
        pl.kernel,
        out_type=jax.ShapeDtypeStruct((p, d), dtype),
        mesh=mesh,
        scratch_types=[
            pltpu.VMEM((nch, _CHUNK), jnp.int32),
            pltpu.VMEM((nch, _CHUNK), jnp.int32),
            pltpu.VMEM((_CHUNK, d), dtype),
            pltpu.VMEM((_CHUNK, d), dtype),
            pltpu.SemaphoreType.DMA,
            pltpu.SemaphoreType.DMA,
            pltpu.SemaphoreType.DMA,
        ],
    )
    def k(hid_hbm, p0_hbm, p1_hbm, out_hbm, i0_v, i1_v, buf0, buf1,
          sem0, sem1, semw):
        wid = lax.axis_index("s") * _NC + lax.axis_index("c")
        base = wid * tok_w
        pltpu.sync_copy(p0_hbm.at[wid], i0_v)
        pltpu.sync_copy(p1_hbm.at[wid], i1_v)

        def rd(c, buf, sem):
            pltpu.async_copy(hid_hbm.at[pl.ds(base + c * _CHUNK, _CHUNK)],
                             buf, sem)

        def rd_wait(buf, sem):
            pltpu.make_async_copy(hid_hbm.at[pl.ds(0, _CHUNK)], buf,
                                  sem).wait()

        def wr(c, buf):
            pltpu.async_copy(buf, out_hbm.at[i0_v.at[c]], semw)
            pltpu.async_copy(buf, out_hbm.at[i1_v.at[c]], semw)

        def wr_wait(buf):
            pltpu.make_async_copy(buf, out_hbm.at[i0_v.at[0]], semw).wait()
            pltpu.make_async_copy(buf, out_hbm.at[i0_v.at[0]], semw).wait()

        rd(0, buf0, sem0)

        def body(i, _):
            e = 2 * i
            rd(e + 1, buf1, sem1)
            rd_wait(buf0, sem0)
            wr(e, buf0)
            wr_wait(buf0)

            @pl.when(e + 2 < nch)
            def _():
                rd(e + 2, buf0, sem0)

            rd_wait(buf1, sem1)
            wr(e + 1, buf1)
            wr_wait(buf1)
            return 0

        lax.fori_loop(0, nch // 2, body, 0)

    return k(hidden_flat, pos0.reshape(_NW, nch, _CHUNK),
             pos1.reshape(_NW, nch, _CHUNK))


def _sc_combine(y, pos0, pos1, wexp0, wexp1, n_tok, d):
    tok_w = n_tok // _NW
    nch = tok_w // _CCH
    nvec = d // _LANES

    mesh = plsc.VectorSubcoreMesh(core_axis_name="c", subcore_axis_name="s")

    @functools.partial(
        pl.kernel,
        out_type=jax.ShapeDtypeStruct((n_tok, d), jnp.float32),
        mesh=mesh,
        scratch_types=[
            pltpu.VMEM((tok_w,), jnp.int32),
            pltpu.VMEM((tok_w,), jnp.int32),
            pltpu.VMEM((tok_w, _LANES), jnp.float32),
            pltpu.VMEM((tok_w, _LANES), jnp.float32),
            pltpu.VMEM((_CCH, d), jnp.float32),
            pltpu.VMEM((_CCH, d), jnp.float32),
            pltpu.VMEM((_CCH, d), jnp.float32),
            pltpu.VMEM((_CCH, d), jnp.float32),
            pltpu.SemaphoreType.DMA,
            pltpu.SemaphoreType.DMA,
            pltpu.SemaphoreType.DMA,
        ],
    )
    def k(y_hbm, p0_hbm, p1_hbm, w0_hbm, w1_hbm, out_hbm,
          i0_v, i1_v, wa_v, wb_v, a0, b0, a1, b1, sem0, sem1, semw):
        wid = lax.axis_index("s") * _NC + lax.axis_index("c")
        base = wid * tok_w
        pltpu.sync_copy(p0_hbm.at[pl.ds(base, tok_w)], i0_v)
        pltpu.sync_copy(p1_hbm.at[pl.ds(base, tok_w)], i1_v)
        pltpu.sync_copy(w0_hbm.at[pl.ds(base, tok_w)], wa_v)
        pltpu.sync_copy(w1_hbm.at[pl.ds(base, tok_w)], wb_v)

        def gath(c, abuf, bbuf, sem):
            pltpu.async_copy(y_hbm.at[i0_v.at[pl.ds(c * _CCH, _CCH)]],
                             abuf, sem)
            pltpu.async_copy(y_hbm.at[i1_v.at[pl.ds(c * _CCH, _CCH)]],
                             bbuf, sem)

        def gath_wait(abuf, bbuf, sem):
            pltpu.make_async_copy(y_hbm.at[i0_v.at[pl.ds(0, _CCH)]],
                                  abuf, sem).wait()
            pltpu.make_async_copy(y_hbm.at[i0_v.at[pl.ds(0, _CCH)]],
                                  bbuf, sem).wait()

        def comp(c, abuf, bbuf):
            was = [wa_v[c * _CCH + r, :] for r in range(_CCH)]
            wbs = [wb_v[c * _CCH + r, :] for r in range(_CCH)]

            def vbody(j, _):
                col = j * _LANES
                for r in range(_CCH):
                    av = abuf[r, pl.ds(col, _LANES)]
                    bv = bbuf[r, pl.ds(col, _LANES)]
                    abuf[r, pl.ds(col, _LANES)] = was[r] * av + wbs[r] * bv
                return 0

            lax.fori_loop(0, nvec, vbody, 0)

        def wrt(c, abuf):
            pltpu.async_copy(abuf, out_hbm.at[pl.ds(base + c * _CCH, _CCH)],
                             semw)

        def wrt_wait(abuf):
            pltpu.make_async_copy(a0, out_hbm.at[pl.ds(0, _CCH)], semw).wait()

        gath(0, a0, b0, sem0)
        gath(1, a1, b1, sem1)

        def body(i, _):
            e = 2 * i
            gath_wait(a0, b0, sem0)
            comp(e, a0, b0)
            wrt(e, a0)

            @pl.when(e + 2 < nch)
            def _():
                wrt_wait(a0)
                gath(e + 2, a0, b0, sem0)

            gath_wait(a1, b1, sem1)
            comp(e + 1, a1, b1)
            wrt(e + 1, a1)

            @pl.when(e + 3 < nch)
            def _():
                wrt_wait(a1)
                gath(e + 3, a1, b1, sem1)
            return 0

        lax.fori_loop(0, nch // 2, body, 0)

        wrt_wait(a0)
        wrt_wait(a1)

    return k(y, pos0, pos1, wexp0, wexp1)


def _matmul_block(be_ref, ff_ref, slot_ref, nxt_ref, x_ref, w_hbm, b_ref,
                  o_ref, wbuf, sems):
    i = pl.program_id(0)
    e_raw = be_ref[i]
    slot = slot_ref[i]

    @pl.when(i == 0)
    def _():
        pltpu.make_async_copy(w_hbm.at[be_ref[0]], wbuf.at[0],
                              sems.at[0]).start()

        @pl.when(nxt_ref[0] < _NUM_EXPERTS)
        def _():
            pltpu.make_async_copy(w_hbm.at[nxt_ref[0]], wbuf.at[1],
                                  sems.at[1]).start()

    @pl.when(ff_ref[i] == 1)
    def _():
        pltpu.make_async_copy(w_hbm.at[0], wbuf.at[slot],
                              sems.at[slot]).wait()

        @pl.when((i > 0) & (nxt_ref[i] < _NUM_EXPERTS))
        def _():
            pltpu.make_async_copy(w_hbm.at[nxt_ref[i]], wbuf.at[1 - slot],
                                  sems.at[1 - slot]).start()

    @pl.when(e_raw < _NUM_EXPERTS)
    def _():
        x = x_ref[...].astype(jnp.bfloat16)
        w = wbuf[slot].astype(jnp.bfloat16)
        y = lax.dot_general(x, w, (((1,), (1,)), ((), ())),
                            preferred_element_type=jnp.float32)
        o_ref[...] = y + b_ref[0]


def _grouped_matmul(dispatch, W, b, block_expert, num_blocks, d):
    e = _NUM_EXPERTS
    be = block_expert
    ff = ((be != jnp.concatenate([jnp.full((1,), 9, jnp.int32), be[:-1]]))
          & (be < e)).astype(jnp.int32)
    slot = ((jnp.cumsum(ff) - 1) % 2).astype(jnp.int32)
    nxt = jnp.min(jnp.where(be[None, :] > be[:, None], be[None, :], e),
                  axis=1).astype(jnp.int32)

    grid_spec = pltpu.PrefetchScalarGridSpec(
        num_scalar_prefetch=4,
        grid=(num_blocks,),
        in_specs=[
            pl.BlockSpec((_BM, d), lambda i, *_: (i, 0)),
            pl.BlockSpec(memory_space=pl.ANY),
            pl.BlockSpec(
                (1, 1, d),
                lambda i, be_, *_: (jnp.minimum(be_[i], e - 1), 0, 0)),
        ],
        out_specs=pl.BlockSpec((_BM, d), lambda i, *_: (i, 0)),
        scratch_shapes=[
            pltpu.VMEM((2, d, d), jnp.float32),
            pltpu.SemaphoreType.DMA((2,)),
        ],
    )
    return pl.pallas_call(
        _matmul_block,
        grid_spec=grid_spec,
        out_shape=jax.ShapeDtypeStruct((num_blocks * _BM, d), jnp.float32),
    )(be, ff, slot, nxt, dispatch, W, b.reshape(b.shape[0], 1, d))


def kernel(hidden, gate_logits, W, b):
    bsz, seq, d = hidden.shape
    n_tok = bsz * seq
    k = _TOP_K
    e = _NUM_EXPERTS
    n_slots = n_tok * k
    p = n_slots + e * _BM
    num_blocks = p // _BM

    hidden_flat = hidden.reshape(n_tok, d)

    g_t = gate_logits.T
    eids = jnp.arange(e, dtype=jnp.int32)[:, None]
    gmax = jnp.max(g_t, axis=0, keepdims=True)
    ex = jnp.exp(g_t - gmax)
    denom = jnp.sum(ex, axis=0, keepdims=True)
    probs_t = ex / denom
    v1 = jnp.max(probs_t, axis=0, keepdims=True)
    e1 = jnp.min(jnp.where(probs_t == v1, eids, e), axis=0)
    probs_m = jnp.where(eids == e1[None, :], -jnp.inf, probs_t)
    v2 = jnp.max(probs_m, axis=0, keepdims=True)
    e2 = jnp.min(jnp.where(probs_m == v2, eids, e), axis=0)

    flat_e = jnp.stack([e1, e2], axis=1).reshape(1, -1)
    oh_t = (eids == flat_e).astype(jnp.int32)
    cum_t = jnp.cumsum(oh_t, axis=1)
    rank = jnp.sum(oh_t * cum_t, axis=0) - 1
    counts = cum_t[:, -1]
    padded_counts = ((counts + _BM - 1) // _BM) * _BM
    padded_end = jnp.cumsum(padded_counts)
    padded_start = padded_end - padded_counts
    seg_base = jnp.sum(oh_t * padded_start[:, None], axis=0)
    padded_pos = (seg_base + rank).astype(jnp.int32)

    block_expert = jnp.sum(
        jnp.arange(num_blocks)[:, None] * _BM >= padded_end[None, :],
        axis=1).astype(jnp.int32)

    pos = padded_pos.reshape(n_tok, k)
    pos0, pos1 = pos[:, 0], pos[:, 1]
    wexp0 = jnp.broadcast_to(v1.reshape(n_tok, 1), (n_tok, _LANES))
    wexp1 = jnp.broadcast_to(v2.reshape(n_tok, 1), (n_tok, _LANES))

    dispatch = _sc_dispatch(hidden_flat, pos0, pos1, n_tok, p, d,
                            jnp.float32)

    y = _grouped_matmul(dispatch, W, b, block_expert, num_blocks, d)

    combined = _sc_combine(y, pos0, pos1, wexp0, wexp1, n_tok, d)
    return combined.reshape(bsz, seq, d)
